# baseline (speedup 1.0000x reference)
import functools

import jax
import jax.numpy as jnp
from jax import lax
from jax.experimental import pallas as pl
from jax.experimental.pallas import tpu as pltpu
from jax.experimental.pallas import tpu_sc as plsc

N_NODES = 10000
D = 128

NC = 2
NS = 16
NW = NC * NS

CHUNK = 128
FRAC0_PM = 500
ROWS_PER_SC = 10112
ROWS_PER_TILE = ROWS_PER_SC // NS
PAD_DST = N_NODES

ZROWS = 64


def _sc_aggregate(total_chunks, hc0, hc1):
    S = -(-max(hc0, hc1) // 16) * 16
    rem = total_chunks - NS * hc0
    mesh = plsc.VectorSubcoreMesh(core_axis_name="c", subcore_axis_name="s")

    @functools.partial(
        pl.kernel,
        mesh=mesh,
        out_type=jax.ShapeDtypeStruct((NC, ROWS_PER_SC, D), jnp.float32),
        scratch_types=[
            pltpu.VMEM_SHARED((ROWS_PER_SC, D), jnp.float32),
            pltpu.VMEM((S, CHUNK), jnp.int32),
            pltpu.VMEM((S, CHUNK), jnp.int32),
            pltpu.VMEM((S,), jnp.int32),
            pltpu.VMEM((CHUNK, D), jnp.float32),
            pltpu.SemaphoreType.DMA,
        ],
    )
    def k(x_hbm, src_hbm, dst_hbm, out_hbm, aggr, src_v, dst_v, rowid_v,
          rows0, sem0):
        cid = lax.axis_index("c")
        sid = lax.axis_index("s")

        start = jnp.where(cid == 0, sid * hc0, NS * hc0 + sid * hc1)
        my_hc = jnp.where(cid == 0, hc0,
                          jnp.clip(rem - sid * hc1, 0, hc1))

        zv = jnp.zeros((16,), jnp.float32)
        for i in range(ZROWS):
            for j in range(D // 16):
                rows0[i, pl.ds(j * 16, 16)] = zv
        my_base = sid * ROWS_PER_TILE
        zsrc = rows0.at[pl.ds(0, ZROWS)]
        for t in range(ROWS_PER_TILE // ZROWS):
            pltpu.sync_copy(zsrc, aggr.at[pl.ds(my_base + t * ZROWS, ZROWS)])
        rzrem = ROWS_PER_TILE % ZROWS
        if rzrem:
            pltpu.sync_copy(
                rows0.at[pl.ds(0, rzrem)],
                aggr.at[pl.ds(my_base + ROWS_PER_TILE - rzrem, rzrem)])
        plsc.subcore_barrier()

        for i in range(S // 16):
            rid = lax.iota(jnp.int32, 16) + (start + i * 16)
            rowid_v[pl.ds(i * 16, 16)] = jnp.minimum(rid, total_chunks - 1)
        pltpu.sync_copy(src_hbm.at[rowid_v], src_v)
        pltpu.sync_copy(dst_hbm.at[rowid_v], dst_v)

        def body(t, carry):
            pltpu.async_copy(x_hbm.at[src_v.at[t]], rows0, sem0).wait()
            pltpu.sync_copy(rows0, aggr.at[dst_v.at[t]], add=True)
            return carry

        lax.fori_loop(0, my_hc, body, 0)
        plsc.subcore_barrier()

        pltpu.sync_copy(aggr.at[pl.ds(my_base, ROWS_PER_TILE)],
                        out_hbm.at[cid].at[pl.ds(my_base, ROWS_PER_TILE)])

    return k


def _mlp_body(x_ref, a_ref, w1_ref, b1_ref, w2_ref, b2_ref, o_ref):
    h = x_ref[...] + a_ref[0] + a_ref[1]
    h = jnp.dot(h, w1_ref[...], preferred_element_type=jnp.float32)
    h = jnp.maximum(h + b1_ref[...], 0.0)
    h = jnp.dot(h, w2_ref[...], preferred_element_type=jnp.float32)
    o_ref[...] = h + b2_ref[...]


def kernel(x, edge_index, W1, b1, W2, b2):
    n = x.shape[0]
    e = edge_index.shape[1]
    src = edge_index[0].astype(jnp.int32)
    dst = edge_index[1].astype(jnp.int32)

    if e % CHUNK:
        npad = CHUNK - e % CHUNK
        pad_dst = (PAD_DST +
                   jnp.arange(npad, dtype=jnp.int32) % (ROWS_PER_SC - PAD_DST))
        src = jnp.concatenate([src, jnp.zeros((npad,), jnp.int32)])
        dst = jnp.concatenate([dst, pad_dst])
    total_chunks = src.shape[0] // CHUNK
    srcc = src.reshape(total_chunks, CHUNK)
    dstc = dst.reshape(total_chunks, CHUNK)

    hc0 = min(-(-total_chunks * FRAC0_PM // (1000 * NS)),
              total_chunks // NS)
    hc1 = -(-max(total_chunks - NS * hc0, 1) // NS)

    partials = _sc_aggregate(total_chunks, hc0, hc1)(x, srcc, dstc)

    rb = 400
    grid = (n // rb,)
    out = pl.pallas_call(
        _mlp_body,
        grid=grid,
        in_specs=[
            pl.BlockSpec((rb, D), lambda j: (j, 0)),
            pl.BlockSpec((NC, rb, D), lambda j: (0, j, 0)),
            pl.BlockSpec((D, D), lambda j: (0, 0)),
            pl.BlockSpec((1, D), lambda j: (0, 0)),
            pl.BlockSpec((D, D), lambda j: (0, 0)),
            pl.BlockSpec((1, D), lambda j: (0, 0)),
        ],
        out_specs=pl.BlockSpec((rb, D), lambda j: (j, 0)),
        out_shape=jax.ShapeDtypeStruct((n, D), jnp.float32),
    )(x, partials, W1, b1.reshape(1, D), W2, b2.reshape(1, D))
    return out


# speedup vs baseline: 1.0540x; 1.0540x over previous
import functools

import jax
import jax.numpy as jnp
from jax import lax
from jax.experimental import pallas as pl
from jax.experimental.pallas import tpu as pltpu
from jax.experimental.pallas import tpu_sc as plsc

N_NODES = 10000
D = 128

NC = 2
NS = 16
NW = NC * NS

CHUNK = 128
FRAC0_PM = 500
ROWS_PER_SC = 10112
ROWS_PER_TILE = ROWS_PER_SC // NS
PAD_DST = N_NODES

ZROWS = 64


def _sc_aggregate(total_chunks, hc0, hc1):
    S = -(-max(hc0, hc1) // 16) * 16
    rem = total_chunks - NS * hc0
    mesh = plsc.VectorSubcoreMesh(core_axis_name="c", subcore_axis_name="s")

    @functools.partial(
        pl.kernel,
        mesh=mesh,
        out_type=jax.ShapeDtypeStruct((NC, ROWS_PER_SC, D), jnp.float32),
        scratch_types=[
            pltpu.VMEM_SHARED((ROWS_PER_SC, D), jnp.float32),
            pltpu.VMEM((S, CHUNK), jnp.int32),
            pltpu.VMEM((S, CHUNK), jnp.int32),
            pltpu.VMEM((S,), jnp.int32),
            pltpu.VMEM((CHUNK, D), jnp.float32),
            pltpu.SemaphoreType.DMA,
        ],
    )
    def k(x_hbm, src_hbm, dst_hbm, out_hbm, aggr, src_v, dst_v, rowid_v,
          rows0, sem0):
        cid = lax.axis_index("c")
        sid = lax.axis_index("s")

        start = jnp.where(cid == 0, sid * hc0, NS * hc0 + sid * hc1)
        my_hc = jnp.where(cid == 0, hc0,
                          jnp.clip(rem - sid * hc1, 0, hc1))

        zv = jnp.zeros((16,), jnp.float32)
        for i in range(ZROWS):
            for j in range(D // 16):
                rows0[i, pl.ds(j * 16, 16)] = zv
        my_base = sid * ROWS_PER_TILE
        zsrc = rows0.at[pl.ds(0, ZROWS)]
        for t in range(ROWS_PER_TILE // ZROWS):
            pltpu.sync_copy(zsrc, aggr.at[pl.ds(my_base + t * ZROWS, ZROWS)])
        rzrem = ROWS_PER_TILE % ZROWS
        if rzrem:
            pltpu.sync_copy(
                rows0.at[pl.ds(0, rzrem)],
                aggr.at[pl.ds(my_base + ROWS_PER_TILE - rzrem, rzrem)])
        plsc.subcore_barrier()

        for i in range(S // 16):
            rid = lax.iota(jnp.int32, 16) + (start + i * 16)
            rowid_v[pl.ds(i * 16, 16)] = jnp.minimum(rid, total_chunks - 1)
        pltpu.sync_copy(src_hbm.at[rowid_v], src_v)
        pltpu.sync_copy(dst_hbm.at[rowid_v], dst_v)

        def body(t, carry):
            pltpu.async_copy(x_hbm.at[src_v.at[t]], rows0, sem0).wait()
            pltpu.sync_copy(rows0, aggr.at[dst_v.at[t]], add=True)
            return carry

        lax.fori_loop(0, my_hc, body, 0)
        plsc.subcore_barrier()

        pltpu.sync_copy(aggr.at[pl.ds(my_base, ROWS_PER_TILE)],
                        out_hbm.at[cid].at[pl.ds(my_base, ROWS_PER_TILE)])

    return k


def _mlp_body(x_ref, a_ref, w1_ref, b1_ref, w2_ref, b2_ref, o_ref):
    h = x_ref[...] + a_ref[0] + a_ref[1]
    h = jnp.dot(h, w1_ref[...], preferred_element_type=jnp.float32)
    h = jnp.maximum(h + b1_ref[...], 0.0)
    h = jnp.dot(h, w2_ref[...], preferred_element_type=jnp.float32)
    o_ref[...] = h + b2_ref[...]


def kernel(x, edge_index, W1, b1, W2, b2):
    n = x.shape[0]
    e = edge_index.shape[1]
    src = edge_index[0].astype(jnp.int32)
    dst = edge_index[1].astype(jnp.int32)

    if e % CHUNK:
        npad = CHUNK - e % CHUNK
        pad_dst = (PAD_DST +
                   jnp.arange(npad, dtype=jnp.int32) % (ROWS_PER_SC - PAD_DST))
        src = jnp.concatenate([src, jnp.zeros((npad,), jnp.int32)])
        dst = jnp.concatenate([dst, pad_dst])
    total_chunks = src.shape[0] // CHUNK
    srcc = src.reshape(total_chunks, CHUNK)
    dstc = dst.reshape(total_chunks, CHUNK)

    hc0 = min(-(-total_chunks * FRAC0_PM // (1000 * NS)),
              total_chunks // NS)
    hc1 = -(-max(total_chunks - NS * hc0, 1) // NS)

    partials = _sc_aggregate(total_chunks, hc0, hc1)(x, srcc, dstc)

    rb = 2000
    grid = (n // rb,)
    out = pl.pallas_call(
        _mlp_body,
        grid=grid,
        in_specs=[
            pl.BlockSpec((rb, D), lambda j: (j, 0)),
            pl.BlockSpec((NC, rb, D), lambda j: (0, j, 0)),
            pl.BlockSpec((D, D), lambda j: (0, 0)),
            pl.BlockSpec((1, D), lambda j: (0, 0)),
            pl.BlockSpec((D, D), lambda j: (0, 0)),
            pl.BlockSpec((1, D), lambda j: (0, 0)),
        ],
        out_specs=pl.BlockSpec((rb, D), lambda j: (j, 0)),
        out_shape=jax.ShapeDtypeStruct((n, D), jnp.float32),
    )(x, partials, W1, b1.reshape(1, D), W2, b2.reshape(1, D))
    return out


# speedup vs baseline: 1.0606x; 1.0062x over previous
import functools

import jax
import jax.numpy as jnp
from jax import lax
from jax.experimental import pallas as pl
from jax.experimental.pallas import tpu as pltpu
from jax.experimental.pallas import tpu_sc as plsc

N_NODES = 10000
D = 128

NC = 2
NS = 16
NW = NC * NS

CHUNK = 128
FRAC0_PM = 500
ROWS_PER_SC = 10112
ROWS_PER_TILE = ROWS_PER_SC // NS
PAD_DST = N_NODES

ZROWS = 64


def _sc_aggregate(total_chunks, hc0, hc1):
    S = -(-max(hc0, hc1) // 16) * 16
    rem = total_chunks - NS * hc0
    mesh = plsc.VectorSubcoreMesh(core_axis_name="c", subcore_axis_name="s")

    @functools.partial(
        pl.kernel,
        mesh=mesh,
        out_type=jax.ShapeDtypeStruct((NC, ROWS_PER_SC, D), jnp.float32),
        scratch_types=[
            pltpu.VMEM_SHARED((ROWS_PER_SC, D), jnp.float32),
            pltpu.VMEM((S, CHUNK), jnp.int32),
            pltpu.VMEM((S, CHUNK), jnp.int32),
            pltpu.VMEM((S,), jnp.int32),
            pltpu.VMEM((CHUNK, D), jnp.float32),
            pltpu.SemaphoreType.DMA,
        ],
    )
    def k(x_hbm, src_hbm, dst_hbm, out_hbm, aggr, src_v, dst_v, rowid_v,
          rows0, sem0):
        cid = lax.axis_index("c")
        sid = lax.axis_index("s")

        start = jnp.where(cid == 0, sid * hc0, NS * hc0 + sid * hc1)
        my_hc = jnp.where(cid == 0, hc0,
                          jnp.clip(rem - sid * hc1, 0, hc1))

        zv = jnp.zeros((16,), jnp.float32)
        for i in range(ZROWS):
            for j in range(D // 16):
                rows0[i, pl.ds(j * 16, 16)] = zv
        my_base = sid * ROWS_PER_TILE
        zsrc = rows0.at[pl.ds(0, ZROWS)]
        for t in range(ROWS_PER_TILE // ZROWS):
            pltpu.sync_copy(zsrc, aggr.at[pl.ds(my_base + t * ZROWS, ZROWS)])
        rzrem = ROWS_PER_TILE % ZROWS
        if rzrem:
            pltpu.sync_copy(
                rows0.at[pl.ds(0, rzrem)],
                aggr.at[pl.ds(my_base + ROWS_PER_TILE - rzrem, rzrem)])
        plsc.subcore_barrier()

        for i in range(S // 16):
            rid = lax.iota(jnp.int32, 16) + (start + i * 16)
            rowid_v[pl.ds(i * 16, 16)] = jnp.minimum(rid, total_chunks - 1)
        pltpu.sync_copy(src_hbm.at[rowid_v], src_v)
        pltpu.sync_copy(dst_hbm.at[rowid_v], dst_v)

        def body(t, carry):
            pltpu.async_copy(x_hbm.at[src_v.at[t]], rows0, sem0).wait()
            pltpu.sync_copy(rows0, aggr.at[dst_v.at[t]], add=True)
            return carry

        lax.fori_loop(0, my_hc, body, 0)
        plsc.subcore_barrier()

        pltpu.sync_copy(aggr.at[pl.ds(my_base, ROWS_PER_TILE)],
                        out_hbm.at[cid].at[pl.ds(my_base, ROWS_PER_TILE)])

    return k


def _mlp_body(x_ref, a_ref, w1_ref, b1_ref, w2_ref, b2_ref, o_ref):
    h = x_ref[...] + a_ref[0] + a_ref[1]
    h = jnp.dot(h, w1_ref[...], preferred_element_type=jnp.float32)
    h = jnp.maximum(h + b1_ref[...], 0.0)
    h = jnp.dot(h, w2_ref[...], preferred_element_type=jnp.float32)
    o_ref[...] = h + b2_ref[...]


def kernel(x, edge_index, W1, b1, W2, b2):
    n = x.shape[0]
    e = edge_index.shape[1]
    src = edge_index[0].astype(jnp.int32)
    dst = edge_index[1].astype(jnp.int32)

    if e % CHUNK:
        npad = CHUNK - e % CHUNK
        pad_dst = (PAD_DST +
                   jnp.arange(npad, dtype=jnp.int32) % (ROWS_PER_SC - PAD_DST))
        src = jnp.concatenate([src, jnp.zeros((npad,), jnp.int32)])
        dst = jnp.concatenate([dst, pad_dst])
    total_chunks = src.shape[0] // CHUNK
    srcc = src.reshape(total_chunks, CHUNK)
    dstc = dst.reshape(total_chunks, CHUNK)

    hc0 = min(-(-total_chunks * FRAC0_PM // (1000 * NS)),
              total_chunks // NS)
    hc1 = -(-max(total_chunks - NS * hc0, 1) // NS)

    partials = _sc_aggregate(total_chunks, hc0, hc1)(x, srcc, dstc)

    rb = 5000
    grid = (n // rb,)
    out = pl.pallas_call(
        _mlp_body,
        grid=grid,
        in_specs=[
            pl.BlockSpec((rb, D), lambda j: (j, 0)),
            pl.BlockSpec((NC, rb, D), lambda j: (0, j, 0)),
            pl.BlockSpec((D, D), lambda j: (0, 0)),
            pl.BlockSpec((1, D), lambda j: (0, 0)),
            pl.BlockSpec((D, D), lambda j: (0, 0)),
            pl.BlockSpec((1, D), lambda j: (0, 0)),
        ],
        out_specs=pl.BlockSpec((rb, D), lambda j: (j, 0)),
        out_shape=jax.ShapeDtypeStruct((n, D), jnp.float32),
    )(x, partials, W1, b1.reshape(1, D), W2, b2.reshape(1, D))
    return out
